# Initial kernel scaffold; baseline (speedup 1.0000x reference)
import jax
import jax.numpy as jnp
from jax.experimental import pallas as pl


def kernel(tokens_ids, weights):
    raise NotImplementedError("write your pallas kernel here")


# speedup vs baseline: 1.8472x; 1.8472x over previous
import functools

import jax
import jax.numpy as jnp
from jax import lax
from jax.experimental import pallas as pl
from jax.experimental.pallas import tpu as pltpu
from jax.experimental.pallas import tpu_sc as plsc

_NUM_CORES = 2
_NUM_SUBCORES = 16
_NUM_WORKERS = _NUM_CORES * _NUM_SUBCORES

_DIM = 64
_GATHER = 128
_J = 8
_SLAB = _GATHER * _J


def _make_gather(num_rows: int):
  assert num_rows % (_NUM_WORKERS * _SLAB) == 0
  slabs_per_worker = num_rows // (_NUM_WORKERS * _SLAB)
  n_idx_rows = num_rows // _GATHER

  mesh = plsc.VectorSubcoreMesh(core_axis_name="c", subcore_axis_name="s")

  @functools.partial(
      pl.kernel,
      mesh=mesh,
      out_type=jax.ShapeDtypeStruct((n_idx_rows, _GATHER, _DIM), jnp.float32),
      scratch_types=[
          pltpu.VMEM((_J, _GATHER), jnp.int32),
          pltpu.VMEM((_J, _GATHER, _DIM), jnp.float32),
          pltpu.SemaphoreType.DMA,
      ],
      compiler_params=pltpu.CompilerParams(use_tc_tiling_on_sc=False),
  )
  def gather_kernel(table_hbm, idx_hbm, out_hbm, idx_v, rows_v, sem):
    wid = lax.axis_index("s") * _NUM_CORES + lax.axis_index("c")

    def slab_body(g, carry):
      row0 = (wid * slabs_per_worker + g) * _J
      pltpu.sync_copy(idx_hbm.at[pl.ds(row0, _J)], idx_v)
      copies = []
      for j in range(_J):
        copies.append(
            pltpu.async_copy(table_hbm.at[idx_v.at[j]], rows_v.at[j], sem))
      for c in copies:
        c.wait()
      pltpu.sync_copy(rows_v, out_hbm.at[pl.ds(row0, _J)])
      return carry

    lax.fori_loop(0, slabs_per_worker, slab_body, 0)

  return gather_kernel


def kernel(tokens_ids, weights):
  b, t = tokens_ids.shape
  num_rows = b * t
  idx = tokens_ids.reshape(num_rows // _GATHER, _GATHER).astype(jnp.int32)
  out = _make_gather(num_rows)(weights, idx)
  return out.reshape(b, t, _DIM)


# speedup vs baseline: 1.8724x; 1.0136x over previous
import functools

import jax
import jax.numpy as jnp
from jax import lax
from jax.experimental import pallas as pl
from jax.experimental.pallas import tpu as pltpu
from jax.experimental.pallas import tpu_sc as plsc

_NUM_CORES = 2
_NUM_SUBCORES = 16
_NUM_WORKERS = _NUM_CORES * _NUM_SUBCORES

_DIM = 64
_GATHER = 128
_J = 5
_SLAB = _GATHER * _J


def _make_gather(num_rows: int):
  assert num_rows % (_NUM_WORKERS * _SLAB) == 0
  slabs_per_worker = num_rows // (_NUM_WORKERS * _SLAB)
  n_idx_rows = num_rows // _GATHER

  mesh = plsc.VectorSubcoreMesh(core_axis_name="c", subcore_axis_name="s")

  @functools.partial(
      pl.kernel,
      mesh=mesh,
      out_type=jax.ShapeDtypeStruct((n_idx_rows, _GATHER, _DIM), jnp.float32),
      scratch_types=[
          pltpu.VMEM((2, _J, _GATHER), jnp.int32),
          pltpu.VMEM((2, _J, _GATHER, _DIM), jnp.float32),
          pltpu.SemaphoreType.DMA,
          pltpu.SemaphoreType.DMA,
          pltpu.SemaphoreType.DMA,
          pltpu.SemaphoreType.DMA,
          pltpu.SemaphoreType.DMA,
          pltpu.SemaphoreType.DMA,
      ],
      compiler_params=pltpu.CompilerParams(use_tc_tiling_on_sc=False),
  )
  def gather_kernel(table_hbm, idx_hbm, out_hbm, idx_v, rows_v,
                    sem_i0, sem_i1, sem_g0, sem_g1, sem_w0, sem_w1):
    wid = lax.axis_index("s") * _NUM_CORES + lax.axis_index("c")
    base = wid * slabs_per_worker * _J

    def idx_rows(g):
      return idx_hbm.at[pl.ds(base + g * _J, _J)]

    def out_rows(g):
      return out_hbm.at[pl.ds(base + g * _J, _J)]

    pltpu.async_copy(idx_rows(0), idx_v.at[0], sem_i0)
    pltpu.async_copy(idx_rows(1), idx_v.at[1], sem_i1)

    def do_slab(g, p, sem_i, sem_g, sem_w):
      idx_p = idx_v.at[p]
      rows_p = rows_v.at[p]
      pltpu.make_async_copy(idx_rows(g), idx_p, sem_i).wait()

      @pl.when(g >= 2)
      def _():
        pltpu.make_async_copy(rows_p, out_rows(g), sem_w).wait()

      copies = []
      for j in range(_J):
        copies.append(
            pltpu.async_copy(table_hbm.at[idx_p.at[j]], rows_p.at[j], sem_g))
      for c in copies:
        c.wait()

      pltpu.async_copy(rows_p, out_rows(g), sem_w)

      @pl.when(g + 2 < slabs_per_worker)
      def _():
        pltpu.async_copy(idx_rows(g + 2), idx_p, sem_i)

    def slab_body(g, carry):
      @pl.when(g % 2 == 0)
      def _():
        do_slab(g, 0, sem_i0, sem_g0, sem_w0)

      @pl.when(g % 2 == 1)
      def _():
        do_slab(g, 1, sem_i1, sem_g1, sem_w1)

      return carry

    lax.fori_loop(0, slabs_per_worker, slab_body, 0)

    last = slabs_per_worker - 1
    pltpu.make_async_copy(rows_v.at[0], out_rows(last - 1), sem_w0).wait()
    pltpu.make_async_copy(rows_v.at[1], out_rows(last), sem_w1).wait()

  return gather_kernel


def kernel(tokens_ids, weights):
  b, t = tokens_ids.shape
  num_rows = b * t
  idx = tokens_ids.reshape(num_rows // _GATHER, _GATHER).astype(jnp.int32)
  out = _make_gather(num_rows)(weights, idx)
  return out.reshape(b, t, _DIM)
